# baseline (speedup 1.0000x reference)
import jax
import jax.numpy as jnp
from jax.experimental import pallas as pl
from jax.experimental.pallas import tpu as pltpu
from jax.experimental.pallas import tpu_sc as plsc

BATCH = 16384
SEQ = 50
EMBED_DIM = 64
VOCAB_ROWS = 100000
NUM_INDICES = BATCH * SEQ
BPW = 16
WINDOW = BPW * SEQ


def kernel(x, table):
    idx = x

    mesh = plsc.VectorSubcoreMesh(core_axis_name="c", subcore_axis_name="s")

    @jax.jit
    def run(table, idx):
        @pl.kernel(
            out_type=jax.ShapeDtypeStruct((BATCH, SEQ, EMBED_DIM), table.dtype),
            mesh=mesh,
            compiler_params=pltpu.CompilerParams(use_tc_tiling_on_sc=False),
        )
        def sc_gather(table_hbm, idx_hbm, out_hbm):
            table_r = table_hbm

            def body(idx_vmem, out_vmem):
                for b in range(BPW):
                    pltpu.sync_copy(
                        table_r.at[idx_vmem.at[b]],
                        out_vmem.at[b],
                    )

            pltpu.emit_pipeline(
                body,
                grid=(BATCH // BPW,),
                in_specs=[
                    pl.BlockSpec((BPW, SEQ), index_map=lambda i: (i, 0))
                ],
                out_specs=[
                    pl.BlockSpec(
                        (BPW, SEQ, EMBED_DIM), index_map=lambda i: (i, 0, 0)
                    )
                ],
                core_axis_name=("c", "s"),
                dimension_semantics=(pltpu.PARALLEL,),
            )(idx_hbm, out_hbm)

        return sc_gather(table, idx)

    return run(table, idx)


# speedup vs baseline: 1.1638x; 1.1638x over previous
import functools
import jax
import jax.numpy as jnp
from jax import lax
from jax.experimental import pallas as pl
from jax.experimental.pallas import tpu as pltpu
from jax.experimental.pallas import tpu_sc as plsc

BATCH = 16384
SEQ = 50
SEQ_PAD = 56
EMBED_DIM = 64
LANE_PAD = 128
NUM_INDICES = BATCH * SEQ
NUM_WORKERS = 32
BPC = 16
CHUNK = BPC * SEQ
BATCH_PER_WORKER = BATCH // NUM_WORKERS
NUM_CHUNKS = BATCH_PER_WORKER // BPC
TC_BLOCK_B = 64


def _sc_gather(table, idx):
    mesh = plsc.VectorSubcoreMesh(core_axis_name="c", subcore_axis_name="s")

    @functools.partial(
        pl.kernel,
        mesh=mesh,
        out_type=jax.ShapeDtypeStruct((BATCH, SEQ_PAD, LANE_PAD), table.dtype),
        scratch_types=[
            pltpu.VMEM((2, CHUNK), jnp.int32),
            pltpu.VMEM((2, CHUNK, EMBED_DIM), jnp.float32),
            pltpu.SemaphoreType.DMA,
            pltpu.SemaphoreType.DMA,
            pltpu.SemaphoreType.DMA,
        ],
        compiler_params=pltpu.CompilerParams(use_tc_tiling_on_sc=False),
    )
    def sc_gather(table_hbm, idx_hbm, out_hbm, idx_v, rows_v, sem_i, sem_g, sem_w):
        wid = lax.axis_index("s") * 2 + lax.axis_index("c")
        base_row = wid * BATCH_PER_WORKER * SEQ
        base_batch = wid * BATCH_PER_WORKER

        def writeback_descs(buf, c):
            descs = []
            for b in range(BPC):
                bid = base_batch + c * BPC + b
                descs.append(
                    pltpu.make_async_copy(
                        rows_v.at[buf, pl.ds(b * SEQ, SEQ)],
                        out_hbm.at[bid, pl.ds(0, SEQ), pl.ds(0, EMBED_DIM)],
                        sem_w,
                    )
                )
            return descs

        pltpu.sync_copy(idx_hbm.at[0, pl.ds(base_row, CHUNK)], idx_v.at[0])

        @pl.loop(0, NUM_CHUNKS // 2)
        def _(g):
            for parity in (0, 1):
                c = g * 2 + parity
                buf = parity
                nxt = 1 - parity
                if parity == 0:
                    pltpu.async_copy(
                        idx_hbm.at[0, pl.ds(base_row + (c + 1) * CHUNK, CHUNK)],
                        idx_v.at[nxt],
                        sem_i,
                    )
                else:
                    @pl.when(g < NUM_CHUNKS // 2 - 1)
                    def _():
                        pltpu.async_copy(
                            idx_hbm.at[0, pl.ds(base_row + (c + 1) * CHUNK, CHUNK)],
                            idx_v.at[nxt],
                            sem_i,
                        )

                @pl.when(g >= 1)
                def _():
                    for d in writeback_descs(buf, c - 2):
                        d.wait()

                pltpu.async_copy(
                    table_hbm.at[idx_v.at[buf]], rows_v.at[buf], sem_g
                ).wait()

                for d in writeback_descs(buf, c):
                    d.start()

                if parity == 0:
                    pltpu.make_async_copy(
                        idx_hbm.at[0, pl.ds(0, CHUNK)], idx_v.at[nxt], sem_i
                    ).wait()
                else:
                    @pl.when(g < NUM_CHUNKS // 2 - 1)
                    def _():
                        pltpu.make_async_copy(
                            idx_hbm.at[0, pl.ds(0, CHUNK)], idx_v.at[nxt], sem_i
                        ).wait()

        for c in (NUM_CHUNKS - 2, NUM_CHUNKS - 1):
            for d in writeback_descs(c % 2, c):
                d.wait()

    return sc_gather(table, idx)


def _tc_repack(mid):
    def body(i_ref, o_ref):
        o_ref[...] = i_ref[:, :SEQ, :EMBED_DIM]

    return pl.pallas_call(
        body,
        out_shape=jax.ShapeDtypeStruct((BATCH, SEQ, EMBED_DIM), mid.dtype),
        grid=(BATCH // TC_BLOCK_B,),
        in_specs=[
            pl.BlockSpec((TC_BLOCK_B, SEQ_PAD, LANE_PAD), lambda i: (i, 0, 0))
        ],
        out_specs=pl.BlockSpec(
            (TC_BLOCK_B, SEQ, EMBED_DIM), lambda i: (i, 0, 0)
        ),
    )(mid)


def kernel(x, table):
    idx = x.reshape(1, NUM_INDICES)

    @jax.jit
    def run(table, idx):
        mid = _sc_gather(table, idx)
        return _tc_repack(mid)

    return run(table, idx)


# speedup vs baseline: 2.4804x; 2.1313x over previous
import functools
import jax
import jax.numpy as jnp
from jax import lax
from jax.experimental import pallas as pl
from jax.experimental.pallas import tpu as pltpu
from jax.experimental.pallas import tpu_sc as plsc

BATCH = 16384
SEQ = 50
SEQ_PAD = 56
EMBED_DIM = 64
LANE_PAD = 128
NUM_INDICES = BATCH * SEQ
NUM_WORKERS = 32
BPC = 16
CHUNK = BPC * SEQ
BATCH_PER_WORKER = BATCH // NUM_WORKERS
NUM_CHUNKS = BATCH_PER_WORKER // BPC
TC_BLOCK_B = 64


def _sc_gather(table, idx):
    mesh = plsc.VectorSubcoreMesh(core_axis_name="c", subcore_axis_name="s")

    @functools.partial(
        pl.kernel,
        mesh=mesh,
        out_type=jax.ShapeDtypeStruct((BATCH, SEQ_PAD, LANE_PAD), table.dtype),
        scratch_types=[
            pltpu.VMEM((2, CHUNK), jnp.int32),
            pltpu.VMEM((2, CHUNK, EMBED_DIM), jnp.float32),
            pltpu.SemaphoreType.DMA,
            pltpu.SemaphoreType.DMA,
            pltpu.SemaphoreType.DMA,
        ],
        compiler_params=pltpu.CompilerParams(use_tc_tiling_on_sc=False),
    )
    def sc_gather(table_hbm, idx_hbm, out_hbm, idx_v, rows_v, sem_i, sem_g, sem_w):
        wid = lax.axis_index("s") * 2 + lax.axis_index("c")
        base_row = wid * BATCH_PER_WORKER * SEQ
        base_batch = wid * BATCH_PER_WORKER

        def writeback_descs(buf, c):
            descs = []
            for b in range(BPC):
                bid = base_batch + c * BPC + b
                descs.append(
                    pltpu.make_async_copy(
                        rows_v.at[buf, pl.ds(b * SEQ, SEQ)],
                        out_hbm.at[bid, pl.ds(0, SEQ), pl.ds(0, EMBED_DIM)],
                        sem_w,
                    )
                )
            return descs

        pltpu.sync_copy(idx_hbm.at[0, pl.ds(base_row, CHUNK)], idx_v.at[0])

        @pl.loop(0, NUM_CHUNKS // 2)
        def _(g):
            for parity in (0, 1):
                c = g * 2 + parity
                buf = parity
                nxt = 1 - parity
                if parity == 0:
                    pltpu.async_copy(
                        idx_hbm.at[0, pl.ds(base_row + (c + 1) * CHUNK, CHUNK)],
                        idx_v.at[nxt],
                        sem_i,
                    )
                else:
                    @pl.when(g < NUM_CHUNKS // 2 - 1)
                    def _():
                        pltpu.async_copy(
                            idx_hbm.at[0, pl.ds(base_row + (c + 1) * CHUNK, CHUNK)],
                            idx_v.at[nxt],
                            sem_i,
                        )

                @pl.when(g >= 1)
                def _():
                    for d in writeback_descs(buf, c - 2):
                        d.wait()

                pltpu.async_copy(
                    table_hbm.at[idx_v.at[buf]], rows_v.at[buf], sem_g
                ).wait()

                for d in writeback_descs(buf, c):
                    d.start()

                if parity == 0:
                    pltpu.make_async_copy(
                        idx_hbm.at[0, pl.ds(0, CHUNK)], idx_v.at[nxt], sem_i
                    ).wait()
                else:
                    @pl.when(g < NUM_CHUNKS // 2 - 1)
                    def _():
                        pltpu.make_async_copy(
                            idx_hbm.at[0, pl.ds(0, CHUNK)], idx_v.at[nxt], sem_i
                        ).wait()

        for c in (NUM_CHUNKS - 2, NUM_CHUNKS - 1):
            for d in writeback_descs(c % 2, c):
                d.wait()

    return sc_gather(table, idx)


def _tc_repack(mid):
    def body(i_ref, o_ref):
        o_ref[...] = i_ref[:, :SEQ, :EMBED_DIM]

    return pl.pallas_call(
        body,
        out_shape=jax.ShapeDtypeStruct((BATCH, SEQ, EMBED_DIM), mid.dtype),
        grid=(BATCH // TC_BLOCK_B,),
        in_specs=[
            pl.BlockSpec((TC_BLOCK_B, SEQ_PAD, LANE_PAD), lambda i: (i, 0, 0))
        ],
        out_specs=pl.BlockSpec(
            (TC_BLOCK_B, SEQ, EMBED_DIM), lambda i: (i, 0, 0)
        ),
    )(mid)


def kernel(x, table):
    idx = x.reshape(1, NUM_INDICES)

    @jax.jit
    def run(table, idx):
        mid = _sc_gather(table, idx)
        return mid[:, :SEQ, :EMBED_DIM]

    return run(table, idx)


# speedup vs baseline: 2.4832x; 1.0011x over previous
import functools
import jax
import jax.numpy as jnp
from jax import lax
from jax.experimental import pallas as pl
from jax.experimental.pallas import tpu as pltpu
from jax.experimental.pallas import tpu_sc as plsc

BATCH = 16384
SEQ = 50
SEQ_PAD = 56
EMBED_DIM = 64
LANE_PAD = 128
NUM_INDICES = BATCH * SEQ
NUM_WORKERS = 32
BPC = 16
CHUNK = BPC * SEQ
BATCH_PER_WORKER = BATCH // NUM_WORKERS
NUM_CHUNKS = BATCH_PER_WORKER // BPC
TC_BLOCK_B = 64


def _sc_gather(table, idx):
    mesh = plsc.VectorSubcoreMesh(core_axis_name="c", subcore_axis_name="s")

    @functools.partial(
        pl.kernel,
        mesh=mesh,
        out_type=jax.ShapeDtypeStruct((BATCH, SEQ_PAD, LANE_PAD), table.dtype),
        scratch_types=[
            pltpu.VMEM((2, CHUNK), jnp.int32),
            pltpu.VMEM((2, CHUNK, EMBED_DIM), jnp.float32),
            pltpu.SemaphoreType.DMA,
            pltpu.SemaphoreType.DMA,
            pltpu.SemaphoreType.DMA,
        ],
        compiler_params=pltpu.CompilerParams(use_tc_tiling_on_sc=False),
    )
    def sc_gather(table_hbm, idx_hbm, out_hbm, idx_v, rows_v, sem_i, sem_g, sem_w):
        wid = lax.axis_index("s") * 2 + lax.axis_index("c")
        base_row = wid * BATCH_PER_WORKER * SEQ
        base_batch = wid * BATCH_PER_WORKER

        def writeback_descs(buf, c):
            descs = []
            for b in range(BPC):
                bid = base_batch + c * BPC + b
                descs.append(
                    pltpu.make_async_copy(
                        rows_v.at[buf, pl.ds(b * SEQ, SEQ)],
                        out_hbm.at[bid, pl.ds(0, SEQ), pl.ds(0, EMBED_DIM)],
                        sem_w,
                    )
                )
            return descs

        pltpu.sync_copy(idx_hbm.at[pl.ds(base_row, CHUNK)], idx_v.at[0])

        @pl.loop(0, NUM_CHUNKS // 2)
        def _(g):
            for parity in (0, 1):
                c = g * 2 + parity
                buf = parity
                nxt = 1 - parity
                if parity == 0:
                    pltpu.async_copy(
                        idx_hbm.at[pl.ds(base_row + (c + 1) * CHUNK, CHUNK)],
                        idx_v.at[nxt],
                        sem_i,
                    )
                else:
                    @pl.when(g < NUM_CHUNKS // 2 - 1)
                    def _():
                        pltpu.async_copy(
                            idx_hbm.at[pl.ds(base_row + (c + 1) * CHUNK, CHUNK)],
                            idx_v.at[nxt],
                            sem_i,
                        )

                @pl.when(g >= 1)
                def _():
                    for d in writeback_descs(buf, c - 2):
                        d.wait()

                pltpu.async_copy(
                    table_hbm.at[idx_v.at[buf]], rows_v.at[buf], sem_g
                ).wait()

                for d in writeback_descs(buf, c):
                    d.start()

                if parity == 0:
                    pltpu.make_async_copy(
                        idx_hbm.at[pl.ds(0, CHUNK)], idx_v.at[nxt], sem_i
                    ).wait()
                else:
                    @pl.when(g < NUM_CHUNKS // 2 - 1)
                    def _():
                        pltpu.make_async_copy(
                            idx_hbm.at[pl.ds(0, CHUNK)], idx_v.at[nxt], sem_i
                        ).wait()

        for c in (NUM_CHUNKS - 2, NUM_CHUNKS - 1):
            for d in writeback_descs(c % 2, c):
                d.wait()

    return sc_gather(table, idx)


def _tc_repack(mid):
    def body(i_ref, o_ref):
        o_ref[...] = i_ref[:, :SEQ, :EMBED_DIM]

    return pl.pallas_call(
        body,
        out_shape=jax.ShapeDtypeStruct((BATCH, SEQ, EMBED_DIM), mid.dtype),
        grid=(BATCH // TC_BLOCK_B,),
        in_specs=[
            pl.BlockSpec((TC_BLOCK_B, SEQ_PAD, LANE_PAD), lambda i: (i, 0, 0))
        ],
        out_specs=pl.BlockSpec(
            (TC_BLOCK_B, SEQ, EMBED_DIM), lambda i: (i, 0, 0)
        ),
    )(mid)


def kernel(x, table):
    idx = x.reshape(NUM_INDICES)

    @jax.jit
    def run(table, idx):
        mid = _sc_gather(table, idx)
        return mid[:, :SEQ, :EMBED_DIM]

    return run(table, idx)
